# baseline (speedup 1.0000x reference)
import functools
import jax
import jax.numpy as jnp
from jax import lax
from jax.experimental import pallas as pl
from jax.experimental.pallas import tpu as pltpu
from jax.experimental.pallas import tpu_sc as plsc

_INFO = plsc.get_sparse_core_info()
_NC = _INFO.num_cores
_NS = _INFO.num_subcores
_NW = _NC * _NS


def _make_sc_kernel(batch, nrows, dim, dtype):
    rows_per_w = nrows // _NW
    mesh = plsc.VectorSubcoreMesh(core_axis_name="c", subcore_axis_name="s")

    @functools.partial(
        pl.kernel,
        mesh=mesh,
        out_type=jax.ShapeDtypeStruct((batch, nrows, dim), dtype),
        scratch_types=[pltpu.SemaphoreType.DMA],
    )
    def k(table_hbm, out_hbm, sem):
        wid = lax.axis_index("s") * _NC + lax.axis_index("c")
        base = wid * rows_per_w
        copies = [
            pltpu.async_copy(
                table_hbm.at[pl.ds(base, rows_per_w)],
                out_hbm.at[b, pl.ds(base, rows_per_w)],
                sem,
            )
            for b in range(batch)
        ]
        for c in copies:
            c.wait()

    return k


def kernel(encoded_tokens, pos_table):
    batch = encoded_tokens.shape[0]
    nrows, dim = pos_table.shape
    return _make_sc_kernel(batch, nrows, dim, pos_table.dtype)(pos_table)


# speedup vs baseline: 31.9434x; 31.9434x over previous
import functools

import jax
import jax.numpy as jnp
from jax import lax
from jax.experimental import pallas as pl
from jax.experimental.pallas import tpu as pltpu
from jax.experimental.pallas import tpu_sc as plsc

_INFO = plsc.get_sparse_core_info()
_NC = _INFO.num_cores
_NS = _INFO.num_subcores
_NW = _NC * _NS


def _make_sc_kernel(batch, nrows, dim, dtype):
    rows_per_w = nrows // _NW
    mesh = plsc.VectorSubcoreMesh(core_axis_name="c", subcore_axis_name="s")

    @functools.partial(
        pl.kernel,
        mesh=mesh,
        out_type=jax.ShapeDtypeStruct((batch, nrows, dim), dtype),
        scratch_types=[
            pltpu.VMEM((rows_per_w, dim), dtype),
            pltpu.SemaphoreType.DMA,
        ],
    )
    def k(table_hbm, out_hbm, rows_v, sem):
        wid = lax.axis_index("s") * _NC + lax.axis_index("c")
        base = wid * rows_per_w
        pltpu.sync_copy(table_hbm.at[pl.ds(base, rows_per_w)], rows_v)
        copies = [
            pltpu.async_copy(rows_v, out_hbm.at[b, pl.ds(base, rows_per_w)], sem)
            for b in range(batch)
        ]
        for c in copies:
            c.wait()

    return k


def kernel(encoded_tokens, pos_table):
    batch = encoded_tokens.shape[0]
    nrows, dim = pos_table.shape
    return _make_sc_kernel(batch, nrows, dim, pos_table.dtype)(pos_table)


# speedup vs baseline: 32.0262x; 1.0026x over previous
import functools

import jax
import jax.numpy as jnp
from jax import lax
from jax.experimental import pallas as pl
from jax.experimental.pallas import tpu as pltpu
from jax.experimental.pallas import tpu_sc as plsc

_INFO = plsc.get_sparse_core_info()
_NC = _INFO.num_cores
_NS = _INFO.num_subcores
_NW = _NC * _NS


def _make_sc_kernel(batch, nrows, dim, dtype):
    rows_per_w = nrows // _NW
    mesh = plsc.VectorSubcoreMesh(core_axis_name="c", subcore_axis_name="s")

    @functools.partial(
        pl.kernel,
        mesh=mesh,
        out_type=jax.ShapeDtypeStruct((batch, nrows, dim), dtype),
        scratch_types=[
            pltpu.VMEM((rows_per_w, dim), dtype),
        ],
    )
    def k(table_hbm, out_hbm, rows_v):
        wid = lax.axis_index("s") * _NC + lax.axis_index("c")
        base = wid * rows_per_w
        pltpu.sync_copy(table_hbm.at[pl.ds(base, rows_per_w)], rows_v)
        for b in range(batch):
            pltpu.sync_copy(rows_v, out_hbm.at[b, pl.ds(base, rows_per_w)])

    return k


def kernel(encoded_tokens, pos_table):
    batch = encoded_tokens.shape[0]
    nrows, dim = pos_table.shape
    return _make_sc_kernel(batch, nrows, dim, pos_table.dtype)(pos_table)
